# baseline (speedup 1.0000x reference)
import jax
import jax.numpy as jnp
from jax.experimental import pallas as pl

N0 = 100000; N1 = 16384; B = 1024
E1 = 262144; E2 = 16384
F0 = 256; H1 = 8; C1 = 256; C2 = 256; NUM_CLASSES = 1000


def kernel(x, edge_index1, edge_index2, W1, att_src1, att_dst1, b1, W2, att_src2, att_dst2, b2, Wl, bl, Wp, bp):
    xs = x[:N1]
    hs = xs @ W1
    hsr = hs.reshape(N1, H1, C1)
    a_src = (hsr * att_src1).sum(-1)
    a_dst = (hsr[:B] * att_dst1).sum(-1)
    a_dst_p = jnp.concatenate([a_dst, jnp.zeros((1, H1), jnp.float32)], axis=0)

    src, dst = edge_index1[0], edge_index1[1]
    rel = dst < B
    dstc = jnp.where(rel, dst, B)
    alpha = jax.nn.leaky_relu(a_src[src] + a_dst_p[dstc], negative_slope=0.2)
    ea = jnp.exp(alpha) * rel[:, None]
    msg = hsr[src] * ea[:, :, None]
    num = jax.ops.segment_sum(msg, dstc, num_segments=B + 1)[:B]
    den = jax.ops.segment_sum(ea, dstc, num_segments=B + 1)[:B]
    out1 = num / (den[:, :, None] + 1e-16)
    h = jax.nn.relu(out1.reshape(B, H1 * C1) + b1)

    hs2 = h @ W2
    a2s = (hs2 * att_src2[0, 0]).sum(-1)
    a2d = (hs2 * att_dst2[0, 0]).sum(-1)
    s2, d2 = edge_index2[0], edge_index2[1]
    alpha2 = jax.nn.leaky_relu(a2s[s2] + a2d[d2], negative_slope=0.2)
    ea2 = jnp.exp(alpha2)
    num2 = jax.ops.segment_sum(hs2[s2] * ea2[:, None], d2, num_segments=B)
    den2 = jax.ops.segment_sum(ea2, d2, num_segments=B)
    h2 = num2 / (den2[:, None] + 1e-16) + b2
    z = jax.nn.relu(h2 @ Wl + bl)
    return z @ Wp + bp


# speedup vs baseline: 41.6846x; 41.6846x over previous
import jax
import jax.numpy as jnp
from jax import lax
from jax.experimental import pallas as pl
from jax.experimental.pallas import tpu as pltpu
from jax.experimental.pallas import tpu_sc as plsc

N1 = 16384; B = 1024
E1 = 262144; E2 = 16384
H1 = 8; C1 = 256; C2 = 256; NUM_CLASSES = 1000

NW = 32
EPW1 = E1 // NW
EPW2 = E2 // NW
WIN = B // NW
CROW = 10240
CPAD = EPW1 + 16
ACC1W = 2064
HSW = 2176
ACC2W = 272
SENT = B


def _iota16():
    return lax.broadcasted_iota(jnp.int32, (16,), 0)


def _splat(buf, i):
    return plsc.load_gather(buf, [jnp.full((16,), i, jnp.int32)])


def _tc1_body(x_ref, w1_ref, avs_ref, avd_ref, hs_ref, adst_ref):
    i = pl.program_id(0)
    acc = jnp.dot(x_ref[...], w1_ref[...], preferred_element_type=jnp.float32)
    ts = acc * avs_ref[...]
    asrc = jnp.concatenate(
        [ts[:, h * C1:(h + 1) * C1].sum(axis=-1, keepdims=True)
         for h in range(H1)], axis=1)
    hs_ref[...] = jnp.concatenate(
        [acc, asrc,
         jnp.zeros((acc.shape[0], HSW - H1 * C1 - H1), jnp.float32)], axis=1)

    @pl.when(i == 0)
    def _():
        td = acc * avd_ref[...]
        adst_ref[...] = jnp.concatenate(
            [td[:B, h * C1:(h + 1) * C1].sum(axis=-1, keepdims=True)
             for h in range(H1)], axis=1)


def _tc1(xs, W1, avs1, avd1):
    bm = 1024
    return pl.pallas_call(
        _tc1_body,
        grid=(N1 // bm,),
        in_specs=[
            pl.BlockSpec((bm, 256), lambda i: (i, 0)),
            pl.BlockSpec((256, H1 * C1), lambda i: (0, 0)),
            pl.BlockSpec((1, H1 * C1), lambda i: (0, 0)),
            pl.BlockSpec((1, H1 * C1), lambda i: (0, 0)),
        ],
        out_specs=[
            pl.BlockSpec((bm, HSW), lambda i: (i, 0)),
            pl.BlockSpec((B, H1), lambda i: (0, 0)),
        ],
        out_shape=[
            jax.ShapeDtypeStruct((N1, HSW), jnp.float32),
            jax.ShapeDtypeStruct((B, H1), jnp.float32),
        ],
    )(xs, W1, avs1, avd1)


def _sca_body(srcE, dstE, cs, cd, counts, st_s, st_d, ls, ld, snt, cbuf):
    c = lax.axis_index("c")
    s = lax.axis_index("s")
    w = s * 2 + c
    pltpu.sync_copy(srcE.at[pl.ds(w * EPW1, EPW1)], st_s)
    pltpu.sync_copy(dstE.at[pl.ds(w * EPW1, EPW1)], st_d)
    for k in range(2048 // 16):
        snt[pl.ds(k * 16, 16)] = jnp.full((16,), SENT, jnp.int32)

    def fbody(j, carry):
        ls[pl.ds(j * 16, 16)] = jnp.zeros((16,), jnp.int32)
        ld[pl.ds(j * 16, 16)] = jnp.full((16,), SENT, jnp.int32)
        return carry

    lax.fori_loop(0, CPAD // 16, fbody, jnp.int32(0))

    def cbody(j, cnt):
        d16 = st_d[pl.ds(j * 16, 16)]
        s16 = st_s[pl.ds(j * 16, 16)]
        m = d16 < B
        mi = m.astype(jnp.int32)
        pos = cnt + plsc.cumsum(mi) - 1
        plsc.store_scatter(ls, [pos], s16, mask=m)
        plsc.store_scatter(ld, [pos], d16, mask=m)
        return cnt + jnp.sum(mi)

    cnt = lax.fori_loop(0, EPW1 // 16, cbody, jnp.int32(0))
    cntp = ((cnt + 15) // 16) * 16
    pltpu.sync_copy(snt, cs.at[pl.ds(w * CROW + 8192, 2048)])
    pltpu.sync_copy(snt, cd.at[pl.ds(w * CROW + 8192, 2048)])
    pltpu.sync_copy(ls, cs.at[pl.ds(w * CROW, CPAD)])
    pltpu.sync_copy(ld, cd.at[pl.ds(w * CROW, CPAD)])
    cbuf[...] = jnp.full((16,), cntp, jnp.int32)
    pltpu.sync_copy(cbuf, counts.at[pl.ds(w * 16, 16)])


def _sca(src1, dst1):
    mesh = plsc.VectorSubcoreMesh(core_axis_name="c", subcore_axis_name="s")
    f = pl.kernel(
        _sca_body,
        out_type=[
            jax.ShapeDtypeStruct((NW * CROW,), jnp.int32),
            jax.ShapeDtypeStruct((NW * CROW,), jnp.int32),
            jax.ShapeDtypeStruct((NW * 16,), jnp.int32),
        ],
        mesh=mesh,
        compiler_params=pltpu.CompilerParams(needs_layout_passes=False),
        scratch_types=[
            pltpu.VMEM((EPW1,), jnp.int32),
            pltpu.VMEM((EPW1,), jnp.int32),
            pltpu.VMEM((CPAD,), jnp.int32),
            pltpu.VMEM((CPAD,), jnp.int32),
            pltpu.VMEM((2048,), jnp.int32),
            pltpu.VMEM((16,), jnp.int32),
        ],
    )
    return f(src1, dst1)


def _scb_body(cs, cd, counts, adst, hs, out_acc,
              st_s, st_d, ls, ld, cnts_l, adst_w, hsbuf, eabuf, didx,
              sidx, acc):
    c = lax.axis_index("c")
    s = lax.axis_index("s")
    iota = _iota16()
    w = s * 2 + c
    lo = w * WIN

    def zbody(j, carry):
        acc[pl.ds(j * 16, 16)] = jnp.zeros((16,), jnp.float32)
        return carry

    lax.fori_loop(0, (WIN * ACC1W) // 16, zbody, jnp.int32(0))
    pltpu.sync_copy(counts, cnts_l)
    pltpu.sync_copy(adst.at[pl.ds(lo, WIN)], adst_w)

    def process_chunk(s16, d16, valid):
        sidx[...] = s16
        pltpu.sync_copy(hs.at[sidx], hsbuf)
        dloc = d16 - lo
        vf = valid.astype(jnp.float32)
        for h in range(H1):
            a_s = plsc.load_gather(
                hsbuf, [_iota16(), jnp.full((16,), H1 * C1 + h, jnp.int32)])
            a_d = plsc.load_gather(
                adst_w, [dloc, jnp.full((16,), h, jnp.int32)])
            al = a_s + a_d
            al = jnp.where(al > 0, al, al * 0.2)
            eabuf[pl.ds(h * 16, 16)] = jnp.exp(al) * vf
        didx[...] = dloc * ACC1W

        def ebody(e, carry):
            dvpi = _splat(didx, e) + _iota16()
            tail = jnp.zeros((16,), jnp.float32)
            for h in range(H1):
                bc = _splat(eabuf, h * 16 + e)
                tail = jnp.where(_iota16() == h, bc, tail)
                for g in range(C1 // 16):
                    o = h * C1 + g * 16
                    val = hsbuf[e, pl.ds(o, 16)] * bc
                    plsc.addupdate_scatter(acc, [dvpi + o], val)
            plsc.addupdate_scatter(acc, [dvpi + H1 * C1], tail)
            return carry

        lax.fori_loop(0, 16, ebody, jnp.int32(0))

    def row_loop(r, carry):
        cr = jnp.max(cnts_l[pl.ds(r * 16, 16)])
        nst = (cr + 2047) // 2048

        def stage_loop(t, carry2):
            pltpu.sync_copy(cs.at[pl.ds(r * CROW + t * 2048, 2048)], st_s)
            pltpu.sync_copy(cd.at[pl.ds(r * CROW + t * 2048, 2048)], st_d)

            def wbody(j, cnt2):
                d16 = st_d[pl.ds(j * 16, 16)]
                s16 = st_s[pl.ds(j * 16, 16)]
                m = (d16 >= lo) & (d16 < lo + WIN)
                mi = m.astype(jnp.int32)
                pos = cnt2 + plsc.cumsum(mi) - 1
                plsc.store_scatter(ls, [pos], s16, mask=m)
                plsc.store_scatter(ld, [pos], d16, mask=m)
                return cnt2 + jnp.sum(mi)

            cnt2 = lax.fori_loop(0, 2048 // 16, wbody, jnp.int32(0))
            plsc.store_scatter(ls, [cnt2 + _iota16()],
                               jnp.zeros((16,), jnp.int32))
            plsc.store_scatter(ld, [cnt2 + _iota16()],
                               jnp.full((16,), lo, jnp.int32))

            def pbody(ci, carry3):
                off = ci * 16
                s16 = ls[pl.ds(off, 16)]
                d16 = ld[pl.ds(off, 16)]
                process_chunk(s16, d16, off + _iota16() < cnt2)
                return carry3

            lax.fori_loop(0, (cnt2 + 15) // 16, pbody, jnp.int32(0))
            return carry2

        lax.fori_loop(0, nst, stage_loop, jnp.int32(0))
        return carry

    lax.fori_loop(0, NW, row_loop, jnp.int32(0))
    pltpu.sync_copy(acc, out_acc.at[pl.ds(w * WIN * ACC1W, WIN * ACC1W)])


def _scb(cs, cd, counts, adstT, hsT):
    mesh = plsc.VectorSubcoreMesh(core_axis_name="c", subcore_axis_name="s")
    f = pl.kernel(
        _scb_body,
        out_type=jax.ShapeDtypeStruct((B * ACC1W,), jnp.float32),
        mesh=mesh,
        compiler_params=pltpu.CompilerParams(needs_layout_passes=False),
        scratch_types=[
            pltpu.VMEM((2048,), jnp.int32),
            pltpu.VMEM((2048,), jnp.int32),
            pltpu.VMEM((2064,), jnp.int32),
            pltpu.VMEM((2064,), jnp.int32),
            pltpu.VMEM((NW * 16,), jnp.int32),
            pltpu.VMEM((WIN, H1), jnp.float32),
            pltpu.VMEM((16, HSW), jnp.float32),
            pltpu.VMEM((H1 * 16,), jnp.float32),
            pltpu.VMEM((16,), jnp.int32),
            pltpu.VMEM((16,), jnp.int32),
            pltpu.VMEM((WIN * ACC1W,), jnp.float32),
        ],
    )
    return f(cs, cd, counts, adstT, hsT)


def _tc2_body(raw_ref, b1_ref, w2_ref, avs2_ref, avd2_ref, hs2_ref, a2_ref):
    raw = raw_ref[...]
    parts = []
    for h in range(H1):
        num = raw[:, h * C1:(h + 1) * C1]
        den = raw[:, H1 * C1 + h:H1 * C1 + h + 1]
        parts.append(num / (den + 1e-16))
    out1 = jnp.concatenate(parts, axis=1)
    hmat = jnp.maximum(out1 + b1_ref[...], 0.0)
    hs2 = jnp.dot(hmat, w2_ref[...], preferred_element_type=jnp.float32)
    hs2_ref[...] = hs2
    t1 = (hs2 * avs2_ref[...]).sum(axis=-1, keepdims=True)
    t2 = (hs2 * avd2_ref[...]).sum(axis=-1, keepdims=True)
    a2_ref[...] = jnp.concatenate([t1, t2], axis=1)


def _tc2(out1raw, b1r, W2, avs2, avd2):
    return pl.pallas_call(
        _tc2_body,
        out_shape=[
            jax.ShapeDtypeStruct((B, C2), jnp.float32),
            jax.ShapeDtypeStruct((B, 2), jnp.float32),
        ],
    )(out1raw, b1r, W2, avs2, avd2)


def _scc_body(srcE, dstE, a2f, hs2, out_acc,
              st_s, st_d, ls, ld, a2_l, h2buf, eabuf, didx, sidx, acc):
    c = lax.axis_index("c")
    s = lax.axis_index("s")
    w = s * 2 + c
    lo = w * WIN

    def zbody(j, carry):
        acc[pl.ds(j * 16, 16)] = jnp.zeros((16,), jnp.float32)
        return carry

    lax.fori_loop(0, (WIN * ACC2W) // 16, zbody, jnp.int32(0))
    pltpu.sync_copy(srcE, st_s)
    pltpu.sync_copy(dstE, st_d)
    pltpu.sync_copy(a2f, a2_l)

    def wbody(j, cnt2):
        d16 = st_d[pl.ds(j * 16, 16)]
        s16 = st_s[pl.ds(j * 16, 16)]
        m = (d16 >= lo) & (d16 < lo + WIN)
        mi = m.astype(jnp.int32)
        pos = cnt2 + plsc.cumsum(mi) - 1
        plsc.store_scatter(ls, [pos], s16, mask=m)
        plsc.store_scatter(ld, [pos], d16, mask=m)
        return cnt2 + jnp.sum(mi)

    cnt2 = lax.fori_loop(0, E2 // 16, wbody, jnp.int32(0))
    plsc.store_scatter(ls, [cnt2 + _iota16()], jnp.zeros((16,), jnp.int32))
    plsc.store_scatter(ld, [cnt2 + _iota16()], jnp.full((16,), lo, jnp.int32))

    def pbody(ci, carry):
        off = ci * 16
        s16 = ls[pl.ds(off, 16)]
        d16 = ld[pl.ds(off, 16)]
        sidx[...] = s16
        pltpu.sync_copy(hs2.at[sidx], h2buf)
        a_s = plsc.load_gather(a2_l, [s16 * 2])
        a_d = plsc.load_gather(a2_l, [d16 * 2 + 1])
        al = a_s + a_d
        al = jnp.where(al > 0, al, al * 0.2)
        vf = (off + _iota16() < cnt2).astype(jnp.float32)
        eabuf[...] = jnp.exp(al) * vf
        didx[...] = (d16 - lo) * ACC2W

        def ebody(e, carry2):
            dvpi = _splat(didx, e) + _iota16()
            bc = _splat(eabuf, e)
            for g in range(C2 // 16):
                val = h2buf[e, pl.ds(g * 16, 16)] * bc
                plsc.addupdate_scatter(acc, [dvpi + g * 16], val)
            plsc.addupdate_scatter(
                acc, [dvpi + C2],
                jnp.where(_iota16() == 0, bc, jnp.zeros((16,), jnp.float32)))
            return carry2

        lax.fori_loop(0, 16, ebody, jnp.int32(0))
        return carry

    lax.fori_loop(0, (cnt2 + 15) // 16, pbody, jnp.int32(0))
    pltpu.sync_copy(acc, out_acc.at[pl.ds(w * WIN * ACC2W, WIN * ACC2W)])


def _scc(src2, dst2, a2f, hs2):
    mesh = plsc.VectorSubcoreMesh(core_axis_name="c", subcore_axis_name="s")
    f = pl.kernel(
        _scc_body,
        out_type=jax.ShapeDtypeStruct((B * ACC2W,), jnp.float32),
        mesh=mesh,
        compiler_params=pltpu.CompilerParams(needs_layout_passes=False),
        scratch_types=[
            pltpu.VMEM((E2,), jnp.int32),
            pltpu.VMEM((E2,), jnp.int32),
            pltpu.VMEM((E2 + 16,), jnp.int32),
            pltpu.VMEM((E2 + 16,), jnp.int32),
            pltpu.VMEM((2 * B,), jnp.float32),
            pltpu.VMEM((16, C2), jnp.float32),
            pltpu.VMEM((16,), jnp.float32),
            pltpu.VMEM((16,), jnp.int32),
            pltpu.VMEM((16,), jnp.int32),
            pltpu.VMEM((WIN * ACC2W,), jnp.float32),
        ],
    )
    return f(src2, dst2, a2f, hs2)


def _tc3_body(raw_ref, b2_ref, wl_ref, bl_ref, wp_ref, bp_ref, o_ref):
    raw = raw_ref[...]
    den = raw[:, C2:C2 + 1]
    h2 = raw[:, :C2] / (den + 1e-16) + b2_ref[...]
    z = jnp.maximum(
        jnp.dot(h2, wl_ref[...], preferred_element_type=jnp.float32)
        + bl_ref[...], 0.0)
    o_ref[...] = (jnp.dot(z, wp_ref[...], preferred_element_type=jnp.float32)
                  + bp_ref[...])


def _tc3(out2raw, b2r, Wl, blr, Wp, bpr):
    return pl.pallas_call(
        _tc3_body,
        out_shape=jax.ShapeDtypeStruct((B, NUM_CLASSES), jnp.float32),
    )(out2raw, b2r, Wl, blr, Wp, bpr)


def kernel(x, edge_index1, edge_index2, W1, att_src1, att_dst1, b1, W2,
           att_src2, att_dst2, b2, Wl, bl, Wp, bp):
    xs = x[:N1]
    avs1 = att_src1.reshape(1, H1 * C1)
    avd1 = att_dst1.reshape(1, H1 * C1)
    hs, adst = _tc1(xs, W1, avs1, avd1)
    cs, cd, counts = _sca(edge_index1[0], edge_index1[1])
    out1raw = _scb(cs, cd, counts, adst, hs)
    hs2, a2I = _tc2(out1raw.reshape(B, ACC1W), b1.reshape(1, -1), W2,
                    att_src2.reshape(1, -1), att_dst2.reshape(1, -1))
    out2raw = _scc(edge_index2[0], edge_index2[1], a2I.reshape(-1), hs2)
    return _tc3(out2raw.reshape(B, ACC2W), b2.reshape(1, -1), Wl,
                bl.reshape(1, -1), Wp, bp.reshape(1, -1))


# speedup vs baseline: 47.3462x; 1.1358x over previous
import jax
import jax.numpy as jnp
from jax import lax
from jax.experimental import pallas as pl
from jax.experimental.pallas import tpu as pltpu
from jax.experimental.pallas import tpu_sc as plsc

N1 = 16384; B = 1024
E1 = 262144; E2 = 16384
H1 = 8; C1 = 256; C2 = 256; NUM_CLASSES = 1000

NW = 32
EPW1 = E1 // NW
EPW2 = E2 // NW
WIN = B // NW
CROW = 10240
CPAD = EPW1 + 16
ACC1W = 2064
HSW = 2176
ACC2W = 272
SENT = B


def _iota16():
    return lax.broadcasted_iota(jnp.int32, (16,), 0)


def _splat(buf, i):
    return plsc.load_gather(buf, [jnp.full((16,), i, jnp.int32)])


def _tc1_body(x_ref, w1_ref, avs_ref, avd_ref, hs_ref, adst_ref):
    i = pl.program_id(0)
    acc = jnp.dot(x_ref[...], w1_ref[...], preferred_element_type=jnp.float32)
    ts = acc * avs_ref[...]
    asrc = jnp.concatenate(
        [ts[:, h * C1:(h + 1) * C1].sum(axis=-1, keepdims=True)
         for h in range(H1)], axis=1)
    hs_ref[...] = jnp.concatenate(
        [acc, asrc,
         jnp.zeros((acc.shape[0], HSW - H1 * C1 - H1), jnp.float32)], axis=1)

    @pl.when(i == 0)
    def _():
        td = acc * avd_ref[...]
        adst_ref[...] = jnp.concatenate(
            [td[:B, h * C1:(h + 1) * C1].sum(axis=-1, keepdims=True)
             for h in range(H1)], axis=1)


def _tc1(xs, W1, avs1, avd1):
    bm = 1024
    return pl.pallas_call(
        _tc1_body,
        grid=(N1 // bm,),
        in_specs=[
            pl.BlockSpec((bm, 256), lambda i: (i, 0)),
            pl.BlockSpec((256, H1 * C1), lambda i: (0, 0)),
            pl.BlockSpec((1, H1 * C1), lambda i: (0, 0)),
            pl.BlockSpec((1, H1 * C1), lambda i: (0, 0)),
        ],
        out_specs=[
            pl.BlockSpec((bm, HSW), lambda i: (i, 0)),
            pl.BlockSpec((B, H1), lambda i: (0, 0)),
        ],
        out_shape=[
            jax.ShapeDtypeStruct((N1, HSW), jnp.float32),
            jax.ShapeDtypeStruct((B, H1), jnp.float32),
        ],
    )(xs, W1, avs1, avd1)


def _sca_body(srcE, dstE, cs, cd, counts, st_s, st_d, ls, ld, snt, cbuf):
    c = lax.axis_index("c")
    s = lax.axis_index("s")
    w = s * 2 + c
    pltpu.sync_copy(srcE.at[pl.ds(w * EPW1, EPW1)], st_s)
    pltpu.sync_copy(dstE.at[pl.ds(w * EPW1, EPW1)], st_d)
    for k in range(2048 // 16):
        snt[pl.ds(k * 16, 16)] = jnp.full((16,), SENT, jnp.int32)

    def fbody(j, carry):
        ls[pl.ds(j * 16, 16)] = jnp.zeros((16,), jnp.int32)
        ld[pl.ds(j * 16, 16)] = jnp.full((16,), SENT, jnp.int32)
        return carry

    lax.fori_loop(0, CPAD // 16, fbody, jnp.int32(0))

    def cbody(j, cnt):
        d16 = st_d[pl.ds(j * 16, 16)]
        s16 = st_s[pl.ds(j * 16, 16)]
        m = d16 < B
        mi = m.astype(jnp.int32)
        pos = cnt + plsc.cumsum(mi) - 1
        plsc.store_scatter(ls, [pos], s16, mask=m)
        plsc.store_scatter(ld, [pos], d16, mask=m)
        return cnt + jnp.sum(mi)

    cnt = lax.fori_loop(0, EPW1 // 16, cbody, jnp.int32(0))
    cntp = ((cnt + 15) // 16) * 16
    pltpu.sync_copy(snt, cs.at[pl.ds(w * CROW + 8192, 2048)])
    pltpu.sync_copy(snt, cd.at[pl.ds(w * CROW + 8192, 2048)])
    pltpu.sync_copy(ls, cs.at[pl.ds(w * CROW, CPAD)])
    pltpu.sync_copy(ld, cd.at[pl.ds(w * CROW, CPAD)])
    cbuf[...] = jnp.full((16,), cntp, jnp.int32)
    pltpu.sync_copy(cbuf, counts.at[pl.ds(w * 16, 16)])


def _sca(src1, dst1):
    mesh = plsc.VectorSubcoreMesh(core_axis_name="c", subcore_axis_name="s")
    f = pl.kernel(
        _sca_body,
        out_type=[
            jax.ShapeDtypeStruct((NW * CROW,), jnp.int32),
            jax.ShapeDtypeStruct((NW * CROW,), jnp.int32),
            jax.ShapeDtypeStruct((NW * 16,), jnp.int32),
        ],
        mesh=mesh,
        compiler_params=pltpu.CompilerParams(needs_layout_passes=False),
        scratch_types=[
            pltpu.VMEM((EPW1,), jnp.int32),
            pltpu.VMEM((EPW1,), jnp.int32),
            pltpu.VMEM((CPAD,), jnp.int32),
            pltpu.VMEM((CPAD,), jnp.int32),
            pltpu.VMEM((2048,), jnp.int32),
            pltpu.VMEM((16,), jnp.int32),
        ],
    )
    return f(src1, dst1)


def _scb_body(cs, cd, counts, adst, hs, out_acc,
              st_s, st_d, ls, ld, cnts_l, adst_w, hsbuf, eabuf, didx,
              sidx, acc):
    c = lax.axis_index("c")
    s = lax.axis_index("s")
    iota = _iota16()
    w = s * 2 + c
    lo = w * WIN

    def zbody(j, carry):
        acc[pl.ds(j * 16, 16)] = jnp.zeros((16,), jnp.float32)
        return carry

    lax.fori_loop(0, (WIN * ACC1W) // 16, zbody, jnp.int32(0))
    pltpu.sync_copy(counts, cnts_l)
    pltpu.sync_copy(adst.at[pl.ds(lo, WIN)], adst_w)

    def process_chunk(s16, d16, nv):
        sidx[...] = s16
        pltpu.sync_copy(hs.at[sidx], hsbuf)
        dloc = d16 - lo
        for h in range(H1):
            a_s = plsc.load_gather(
                hsbuf, [_iota16(), jnp.full((16,), H1 * C1 + h, jnp.int32)])
            a_d = plsc.load_gather(
                adst_w, [dloc, jnp.full((16,), h, jnp.int32)])
            al = a_s + a_d
            al = jnp.where(al > 0, al, al * 0.2)
            eabuf[pl.ds(h * 16, 16)] = jnp.exp(al)

        def ebody(e, carry):
            dbase = jnp.sum(jnp.where(_iota16() == e, dloc, 0)) * ACC1W
            tail = jnp.zeros((16,), jnp.float32)
            for h in range(H1):
                bc = _splat(eabuf, h * 16 + e)
                tail = jnp.where(_iota16() == h, bc, tail)
                for g in range(C1 // 16):
                    o = h * C1 + g * 16
                    val = hsbuf[e, pl.ds(o, 16)] * bc
                    plsc.addupdate(acc.at[pl.ds(dbase + o, 16)], val)
            plsc.addupdate(acc.at[pl.ds(dbase + H1 * C1, 16)], tail)
            return carry

        lax.fori_loop(0, nv, ebody, jnp.int32(0))

    def row_loop(r, carry):
        cr = jnp.max(cnts_l[pl.ds(r * 16, 16)])
        nst = (cr + 2047) // 2048

        def stage_loop(t, carry2):
            pltpu.sync_copy(cs.at[pl.ds(r * CROW + t * 2048, 2048)], st_s)
            pltpu.sync_copy(cd.at[pl.ds(r * CROW + t * 2048, 2048)], st_d)

            def wbody(j, cnt2):
                d16 = st_d[pl.ds(j * 16, 16)]
                s16 = st_s[pl.ds(j * 16, 16)]
                m = (d16 >= lo) & (d16 < lo + WIN)
                mi = m.astype(jnp.int32)
                pos = cnt2 + plsc.cumsum(mi) - 1
                plsc.store_scatter(ls, [pos], s16, mask=m)
                plsc.store_scatter(ld, [pos], d16, mask=m)
                return cnt2 + jnp.sum(mi)

            cnt2 = lax.fori_loop(0, 2048 // 16, wbody, jnp.int32(0))
            plsc.store_scatter(ls, [cnt2 + _iota16()],
                               jnp.zeros((16,), jnp.int32))
            plsc.store_scatter(ld, [cnt2 + _iota16()],
                               jnp.full((16,), lo, jnp.int32))

            def pbody(ci, carry3):
                off = ci * 16
                s16 = ls[pl.ds(off, 16)]
                d16 = ld[pl.ds(off, 16)]
                process_chunk(s16, d16, jnp.minimum(cnt2 - off, 16))
                return carry3

            lax.fori_loop(0, (cnt2 + 15) // 16, pbody, jnp.int32(0))
            return carry2

        lax.fori_loop(0, nst, stage_loop, jnp.int32(0))
        return carry

    lax.fori_loop(0, NW, row_loop, jnp.int32(0))
    pltpu.sync_copy(acc, out_acc.at[pl.ds(w * WIN * ACC1W, WIN * ACC1W)])


def _scb(cs, cd, counts, adstT, hsT):
    mesh = plsc.VectorSubcoreMesh(core_axis_name="c", subcore_axis_name="s")
    f = pl.kernel(
        _scb_body,
        out_type=jax.ShapeDtypeStruct((B * ACC1W,), jnp.float32),
        mesh=mesh,
        compiler_params=pltpu.CompilerParams(needs_layout_passes=False),
        scratch_types=[
            pltpu.VMEM((2048,), jnp.int32),
            pltpu.VMEM((2048,), jnp.int32),
            pltpu.VMEM((2064,), jnp.int32),
            pltpu.VMEM((2064,), jnp.int32),
            pltpu.VMEM((NW * 16,), jnp.int32),
            pltpu.VMEM((WIN, H1), jnp.float32),
            pltpu.VMEM((16, HSW), jnp.float32),
            pltpu.VMEM((H1 * 16,), jnp.float32),
            pltpu.VMEM((16,), jnp.int32),
            pltpu.VMEM((16,), jnp.int32),
            pltpu.VMEM((WIN * ACC1W,), jnp.float32),
        ],
    )
    return f(cs, cd, counts, adstT, hsT)


def _tc2_body(raw_ref, b1_ref, w2_ref, avs2_ref, avd2_ref, hs2_ref, a2_ref):
    raw = raw_ref[...]
    parts = []
    for h in range(H1):
        num = raw[:, h * C1:(h + 1) * C1]
        den = raw[:, H1 * C1 + h:H1 * C1 + h + 1]
        parts.append(num / (den + 1e-16))
    out1 = jnp.concatenate(parts, axis=1)
    hmat = jnp.maximum(out1 + b1_ref[...], 0.0)
    hs2 = jnp.dot(hmat, w2_ref[...], preferred_element_type=jnp.float32)
    hs2_ref[...] = hs2
    t1 = (hs2 * avs2_ref[...]).sum(axis=-1, keepdims=True)
    t2 = (hs2 * avd2_ref[...]).sum(axis=-1, keepdims=True)
    a2_ref[...] = jnp.concatenate([t1, t2], axis=1)


def _tc2(out1raw, b1r, W2, avs2, avd2):
    return pl.pallas_call(
        _tc2_body,
        out_shape=[
            jax.ShapeDtypeStruct((B, C2), jnp.float32),
            jax.ShapeDtypeStruct((B, 2), jnp.float32),
        ],
    )(out1raw, b1r, W2, avs2, avd2)


def _scc_body(srcE, dstE, a2f, hs2, out_acc,
              st_s, st_d, ls, ld, a2_l, h2buf, eabuf, didx, sidx, acc):
    c = lax.axis_index("c")
    s = lax.axis_index("s")
    w = s * 2 + c
    lo = w * WIN

    def zbody(j, carry):
        acc[pl.ds(j * 16, 16)] = jnp.zeros((16,), jnp.float32)
        return carry

    lax.fori_loop(0, (WIN * ACC2W) // 16, zbody, jnp.int32(0))
    pltpu.sync_copy(srcE, st_s)
    pltpu.sync_copy(dstE, st_d)
    pltpu.sync_copy(a2f, a2_l)

    def wbody(j, cnt2):
        d16 = st_d[pl.ds(j * 16, 16)]
        s16 = st_s[pl.ds(j * 16, 16)]
        m = (d16 >= lo) & (d16 < lo + WIN)
        mi = m.astype(jnp.int32)
        pos = cnt2 + plsc.cumsum(mi) - 1
        plsc.store_scatter(ls, [pos], s16, mask=m)
        plsc.store_scatter(ld, [pos], d16, mask=m)
        return cnt2 + jnp.sum(mi)

    cnt2 = lax.fori_loop(0, E2 // 16, wbody, jnp.int32(0))
    plsc.store_scatter(ls, [cnt2 + _iota16()], jnp.zeros((16,), jnp.int32))
    plsc.store_scatter(ld, [cnt2 + _iota16()], jnp.full((16,), lo, jnp.int32))

    def pbody(ci, carry):
        off = ci * 16
        s16 = ls[pl.ds(off, 16)]
        d16 = ld[pl.ds(off, 16)]
        sidx[...] = s16
        pltpu.sync_copy(hs2.at[sidx], h2buf)
        a_s = plsc.load_gather(a2_l, [s16 * 2])
        a_d = plsc.load_gather(a2_l, [d16 * 2 + 1])
        al = a_s + a_d
        al = jnp.where(al > 0, al, al * 0.2)
        eabuf[...] = jnp.exp(al)
        dloc = d16 - lo

        def ebody(e, carry2):
            dbase = jnp.sum(jnp.where(_iota16() == e, dloc, 0)) * ACC2W
            bc = _splat(eabuf, e)
            for g in range(C2 // 16):
                val = h2buf[e, pl.ds(g * 16, 16)] * bc
                plsc.addupdate(acc.at[pl.ds(dbase + g * 16, 16)], val)
            plsc.addupdate(
                acc.at[pl.ds(dbase + C2, 16)],
                jnp.where(_iota16() == 0, bc, jnp.zeros((16,), jnp.float32)))
            return carry2

        lax.fori_loop(0, jnp.minimum(cnt2 - off, 16), ebody, jnp.int32(0))
        return carry

    lax.fori_loop(0, (cnt2 + 15) // 16, pbody, jnp.int32(0))
    pltpu.sync_copy(acc, out_acc.at[pl.ds(w * WIN * ACC2W, WIN * ACC2W)])


def _scc(src2, dst2, a2f, hs2):
    mesh = plsc.VectorSubcoreMesh(core_axis_name="c", subcore_axis_name="s")
    f = pl.kernel(
        _scc_body,
        out_type=jax.ShapeDtypeStruct((B * ACC2W,), jnp.float32),
        mesh=mesh,
        compiler_params=pltpu.CompilerParams(needs_layout_passes=False),
        scratch_types=[
            pltpu.VMEM((E2,), jnp.int32),
            pltpu.VMEM((E2,), jnp.int32),
            pltpu.VMEM((E2 + 16,), jnp.int32),
            pltpu.VMEM((E2 + 16,), jnp.int32),
            pltpu.VMEM((2 * B,), jnp.float32),
            pltpu.VMEM((16, C2), jnp.float32),
            pltpu.VMEM((16,), jnp.float32),
            pltpu.VMEM((16,), jnp.int32),
            pltpu.VMEM((16,), jnp.int32),
            pltpu.VMEM((WIN * ACC2W,), jnp.float32),
        ],
    )
    return f(src2, dst2, a2f, hs2)


def _tc3_body(raw_ref, b2_ref, wl_ref, bl_ref, wp_ref, bp_ref, o_ref):
    raw = raw_ref[...]
    den = raw[:, C2:C2 + 1]
    h2 = raw[:, :C2] / (den + 1e-16) + b2_ref[...]
    z = jnp.maximum(
        jnp.dot(h2, wl_ref[...], preferred_element_type=jnp.float32)
        + bl_ref[...], 0.0)
    o_ref[...] = (jnp.dot(z, wp_ref[...], preferred_element_type=jnp.float32)
                  + bp_ref[...])


def _tc3(out2raw, b2r, Wl, blr, Wp, bpr):
    return pl.pallas_call(
        _tc3_body,
        out_shape=jax.ShapeDtypeStruct((B, NUM_CLASSES), jnp.float32),
    )(out2raw, b2r, Wl, blr, Wp, bpr)


def kernel(x, edge_index1, edge_index2, W1, att_src1, att_dst1, b1, W2,
           att_src2, att_dst2, b2, Wl, bl, Wp, bp):
    xs = x[:N1]
    avs1 = att_src1.reshape(1, H1 * C1)
    avd1 = att_dst1.reshape(1, H1 * C1)
    hs, adst = _tc1(xs, W1, avs1, avd1)
    cs, cd, counts = _sca(edge_index1[0], edge_index1[1])
    out1raw = _scb(cs, cd, counts, adst, hs)
    hs2, a2I = _tc2(out1raw.reshape(B, ACC1W), b1.reshape(1, -1), W2,
                    att_src2.reshape(1, -1), att_dst2.reshape(1, -1))
    out2raw = _scc(edge_index2[0], edge_index2[1], a2I.reshape(-1), hs2)
    return _tc3(out2raw.reshape(B, ACC2W), b2.reshape(1, -1), Wl,
                bl.reshape(1, -1), Wp, bp.reshape(1, -1))


# speedup vs baseline: 48.5756x; 1.0260x over previous
import jax
import jax.numpy as jnp
from jax import lax
from jax.experimental import pallas as pl
from jax.experimental.pallas import tpu as pltpu
from jax.experimental.pallas import tpu_sc as plsc

N1 = 16384; B = 1024
E1 = 262144; E2 = 16384
H1 = 8; C1 = 256; C2 = 256; NUM_CLASSES = 1000

NW = 32
EPW1 = E1 // NW
EPW2 = E2 // NW
WIN = B // NW
CROW = 10240
CPAD = EPW1 + 16
ACC1W = 2064
HSW = 2176
ACC2W = 272
SENT = B


def _iota16():
    return lax.broadcasted_iota(jnp.int32, (16,), 0)


def _splat(buf, i):
    return plsc.load_gather(buf, [jnp.full((16,), i, jnp.int32)])


def _tc1_body(x_ref, w1_ref, avs_ref, avd_ref, hs_ref, adst_ref):
    i = pl.program_id(0)
    acc = jnp.dot(x_ref[...], w1_ref[...], preferred_element_type=jnp.float32)
    ts = acc * avs_ref[...]
    asrc = jnp.concatenate(
        [ts[:, h * C1:(h + 1) * C1].sum(axis=-1, keepdims=True)
         for h in range(H1)], axis=1)
    hs_ref[...] = jnp.concatenate(
        [acc, asrc,
         jnp.zeros((acc.shape[0], HSW - H1 * C1 - H1), jnp.float32)], axis=1)

    @pl.when(i == 0)
    def _():
        td = acc * avd_ref[...]
        adst_ref[...] = jnp.concatenate(
            [td[:B, h * C1:(h + 1) * C1].sum(axis=-1, keepdims=True)
             for h in range(H1)], axis=1)


def _tc1(xs, W1, avs1, avd1):
    bm = 1024
    return pl.pallas_call(
        _tc1_body,
        grid=(N1 // bm,),
        in_specs=[
            pl.BlockSpec((bm, 256), lambda i: (i, 0)),
            pl.BlockSpec((256, H1 * C1), lambda i: (0, 0)),
            pl.BlockSpec((1, H1 * C1), lambda i: (0, 0)),
            pl.BlockSpec((1, H1 * C1), lambda i: (0, 0)),
        ],
        out_specs=[
            pl.BlockSpec((bm, HSW), lambda i: (i, 0)),
            pl.BlockSpec((B, H1), lambda i: (0, 0)),
        ],
        out_shape=[
            jax.ShapeDtypeStruct((N1, HSW), jnp.float32),
            jax.ShapeDtypeStruct((B, H1), jnp.float32),
        ],
    )(xs, W1, avs1, avd1)


def _sca_body(srcE, dstE, cs, cd, counts, st_s, st_d, ls, ld, snt, cbuf):
    c = lax.axis_index("c")
    s = lax.axis_index("s")
    w = s * 2 + c
    pltpu.sync_copy(srcE.at[pl.ds(w * EPW1, EPW1)], st_s)
    pltpu.sync_copy(dstE.at[pl.ds(w * EPW1, EPW1)], st_d)
    for k in range(2048 // 16):
        snt[pl.ds(k * 16, 16)] = jnp.full((16,), SENT, jnp.int32)

    def fbody(j, carry):
        ls[pl.ds(j * 16, 16)] = jnp.zeros((16,), jnp.int32)
        ld[pl.ds(j * 16, 16)] = jnp.full((16,), SENT, jnp.int32)
        return carry

    lax.fori_loop(0, CPAD // 16, fbody, jnp.int32(0))

    def cbody(j, cnt):
        d16 = st_d[pl.ds(j * 16, 16)]
        s16 = st_s[pl.ds(j * 16, 16)]
        m = d16 < B
        mi = m.astype(jnp.int32)
        pos = cnt + plsc.cumsum(mi) - 1
        plsc.store_scatter(ls, [pos], s16, mask=m)
        plsc.store_scatter(ld, [pos], d16, mask=m)
        return cnt + jnp.sum(mi)

    cnt = lax.fori_loop(0, EPW1 // 16, cbody, jnp.int32(0))
    cntp = ((cnt + 15) // 16) * 16
    pltpu.sync_copy(snt, cs.at[pl.ds(w * CROW + 8192, 2048)])
    pltpu.sync_copy(snt, cd.at[pl.ds(w * CROW + 8192, 2048)])
    pltpu.sync_copy(ls, cs.at[pl.ds(w * CROW, CPAD)])
    pltpu.sync_copy(ld, cd.at[pl.ds(w * CROW, CPAD)])
    cbuf[...] = jnp.full((16,), cntp, jnp.int32)
    pltpu.sync_copy(cbuf, counts.at[pl.ds(w * 16, 16)])


def _sca(src1, dst1):
    mesh = plsc.VectorSubcoreMesh(core_axis_name="c", subcore_axis_name="s", num_cores=2, num_subcores=16)
    f = pl.kernel(
        _sca_body,
        out_type=[
            jax.ShapeDtypeStruct((NW * CROW,), jnp.int32),
            jax.ShapeDtypeStruct((NW * CROW,), jnp.int32),
            jax.ShapeDtypeStruct((NW * 16,), jnp.int32),
        ],
        mesh=mesh,
        compiler_params=pltpu.CompilerParams(needs_layout_passes=False),
        scratch_types=[
            pltpu.VMEM((EPW1,), jnp.int32),
            pltpu.VMEM((EPW1,), jnp.int32),
            pltpu.VMEM((CPAD,), jnp.int32),
            pltpu.VMEM((CPAD,), jnp.int32),
            pltpu.VMEM((2048,), jnp.int32),
            pltpu.VMEM((16,), jnp.int32),
        ],
    )
    return f(src1, dst1)


def _scb_body(cs, cd, counts, adst, hs, out_acc,
              st_s0, st_d0, st_s1, st_d1, ls, ld, cnts_l, adst_w, hsbuf,
              eabuf, sidx, acc, sem_s0, sem_d0, sem_s1, sem_d1):
    c = lax.axis_index("c")
    s = lax.axis_index("s")
    w = s * 2 + c
    lo = w * WIN

    def zbody(j, carry):
        acc[pl.ds(j * 16, 16)] = jnp.zeros((16,), jnp.float32)
        return carry

    lax.fori_loop(0, (WIN * ACC1W) // 16, zbody, jnp.int32(0))
    pltpu.sync_copy(counts, cnts_l)
    pltpu.sync_copy(adst.at[pl.ds(lo, WIN)], adst_w)

    def process_chunk(s16, d16, nv):
        vmask = _iota16() < nv
        s16 = jnp.where(vmask, s16, 0)
        d16 = jnp.where(vmask, d16, lo)
        sidx[...] = s16
        pltpu.sync_copy(hs.at[sidx], hsbuf)
        dloc = d16 - lo
        for h in range(H1):
            a_s = plsc.load_gather(
                hsbuf, [_iota16(), jnp.full((16,), H1 * C1 + h, jnp.int32)])
            a_d = plsc.load_gather(
                adst_w, [dloc, jnp.full((16,), h, jnp.int32)])
            al = a_s + a_d
            al = jnp.where(al > 0, al, al * 0.2)
            eabuf[pl.ds(h * 16, 16)] = jnp.exp(al)

        def ebody(e, carry):
            dbase = jnp.sum(jnp.where(_iota16() == e, dloc, 0)) * ACC1W
            tail = jnp.zeros((16,), jnp.float32)
            for h in range(H1):
                bc = _splat(eabuf, h * 16 + e)
                tail = jnp.where(_iota16() == h, bc, tail)
                for g in range(C1 // 16):
                    o = h * C1 + g * 16
                    val = hsbuf[e, pl.ds(o, 16)] * bc
                    plsc.addupdate(acc.at[pl.ds(dbase + o, 16)], val)
            plsc.addupdate(acc.at[pl.ds(dbase + H1 * C1, 16)], tail)
            return carry

        lax.fori_loop(0, nv, ebody, jnp.int32(0))

    def scan_stage(st_s, st_d):
        def wbody(j, cnt2):
            d16 = st_d[pl.ds(j * 16, 16)]
            s16 = st_s[pl.ds(j * 16, 16)]
            m = (d16 >= lo) & (d16 < lo + WIN)
            mi = m.astype(jnp.int32)
            pos = cnt2 + plsc.cumsum(mi) - 1
            plsc.store_scatter(ls, [pos], s16, mask=m)
            plsc.store_scatter(ld, [pos], d16, mask=m)
            return cnt2 + jnp.sum(mi)

        cnt2 = lax.fori_loop(0, 2048 // 16, wbody, jnp.int32(0))
        plsc.store_scatter(ls, [cnt2 + _iota16()],
                           jnp.zeros((16,), jnp.int32))
        plsc.store_scatter(ld, [cnt2 + _iota16()],
                           jnp.full((16,), lo, jnp.int32))

        def pbody(ci, carry3):
            off = ci * 16
            process_chunk(ls[pl.ds(off, 16)], ld[pl.ds(off, 16)],
                          jnp.minimum(cnt2 - off, 16))
            return carry3

        lax.fori_loop(0, (cnt2 + 15) // 16, pbody, jnp.int32(0))

    def do_row(r, st_s, st_d):
        scan_stage(st_s, st_d)
        cr = jnp.max(cnts_l[pl.ds(r * 16, 16)])
        nst = (cr + 2047) // 2048

        def extra(t, carry2):
            pltpu.sync_copy(cs.at[pl.ds(r * CROW + (t + 1) * 2048, 2048)],
                            st_s)
            pltpu.sync_copy(cd.at[pl.ds(r * CROW + (t + 1) * 2048, 2048)],
                            st_d)
            scan_stage(st_s, st_d)
            return carry2

        lax.fori_loop(0, nst - 1, extra, jnp.int32(0))

    def start_row(r, st_s, st_d, sem_s, sem_d):
        rr = jnp.minimum(r, NW - 1)
        pltpu.async_copy(cs.at[pl.ds(rr * CROW, 2048)], st_s, sem_s)
        pltpu.async_copy(cd.at[pl.ds(rr * CROW, 2048)], st_d, sem_d)

    def wait_row(st_s, st_d, sem_s, sem_d):
        pltpu.make_async_copy(cs.at[pl.ds(0, 2048)], st_s, sem_s).wait()
        pltpu.make_async_copy(cd.at[pl.ds(0, 2048)], st_d, sem_d).wait()

    start_row(jnp.int32(0), st_s0, st_d0, sem_s0, sem_d0)

    def pair_loop(i, carry):
        r0 = i * 2
        wait_row(st_s0, st_d0, sem_s0, sem_d0)
        start_row(r0 + 1, st_s1, st_d1, sem_s1, sem_d1)
        do_row(r0, st_s0, st_d0)
        wait_row(st_s1, st_d1, sem_s1, sem_d1)
        start_row(r0 + 2, st_s0, st_d0, sem_s0, sem_d0)
        do_row(r0 + 1, st_s1, st_d1)
        return carry

    lax.fori_loop(0, NW // 2, pair_loop, jnp.int32(0))
    wait_row(st_s0, st_d0, sem_s0, sem_d0)
    pltpu.sync_copy(acc, out_acc.at[pl.ds(w * WIN * ACC1W, WIN * ACC1W)])


def _scb(cs, cd, counts, adstT, hsT):
    mesh = plsc.VectorSubcoreMesh(core_axis_name="c", subcore_axis_name="s", num_cores=2, num_subcores=16)
    f = pl.kernel(
        _scb_body,
        out_type=jax.ShapeDtypeStruct((B * ACC1W,), jnp.float32),
        mesh=mesh,
        compiler_params=pltpu.CompilerParams(needs_layout_passes=False),
        scratch_types=[
            pltpu.VMEM((2048,), jnp.int32),
            pltpu.VMEM((2048,), jnp.int32),
            pltpu.VMEM((2048,), jnp.int32),
            pltpu.VMEM((2048,), jnp.int32),
            pltpu.VMEM((2064 + 16,), jnp.int32),
            pltpu.VMEM((2064 + 16,), jnp.int32),
            pltpu.VMEM((NW * 16,), jnp.int32),
            pltpu.VMEM((WIN, H1), jnp.float32),
            pltpu.VMEM((16, HSW), jnp.float32),
            pltpu.VMEM((H1 * 16,), jnp.float32),
            pltpu.VMEM((16,), jnp.int32),
            pltpu.VMEM((WIN * ACC1W,), jnp.float32),
            pltpu.SemaphoreType.DMA,
            pltpu.SemaphoreType.DMA,
            pltpu.SemaphoreType.DMA,
            pltpu.SemaphoreType.DMA,
        ],
    )
    return f(cs, cd, counts, adstT, hsT)


def _tc2_body(raw_ref, b1_ref, w2_ref, avs2_ref, avd2_ref, hs2_ref, a2_ref):
    raw = raw_ref[...]
    parts = []
    for h in range(H1):
        num = raw[:, h * C1:(h + 1) * C1]
        den = raw[:, H1 * C1 + h:H1 * C1 + h + 1]
        parts.append(num / (den + 1e-16))
    out1 = jnp.concatenate(parts, axis=1)
    hmat = jnp.maximum(out1 + b1_ref[...], 0.0)
    hs2 = jnp.dot(hmat, w2_ref[...], preferred_element_type=jnp.float32)
    hs2_ref[...] = hs2
    t1 = (hs2 * avs2_ref[...]).sum(axis=-1, keepdims=True)
    t2 = (hs2 * avd2_ref[...]).sum(axis=-1, keepdims=True)
    a2_ref[...] = jnp.concatenate([t1, t2], axis=1)


def _tc2(out1raw, b1r, W2, avs2, avd2):
    return pl.pallas_call(
        _tc2_body,
        out_shape=[
            jax.ShapeDtypeStruct((B, C2), jnp.float32),
            jax.ShapeDtypeStruct((B, 2), jnp.float32),
        ],
    )(out1raw, b1r, W2, avs2, avd2)


def _scc_body(srcE, dstE, a2f, hs2, out_acc,
              st_s, st_d, ls, ld, a2_l, h2buf, eabuf, didx, sidx, acc):
    c = lax.axis_index("c")
    s = lax.axis_index("s")
    w = s * 2 + c
    lo = w * WIN

    def zbody(j, carry):
        acc[pl.ds(j * 16, 16)] = jnp.zeros((16,), jnp.float32)
        return carry

    lax.fori_loop(0, (WIN * ACC2W) // 16, zbody, jnp.int32(0))
    pltpu.sync_copy(srcE, st_s)
    pltpu.sync_copy(dstE, st_d)
    pltpu.sync_copy(a2f, a2_l)

    def wbody(j, cnt2):
        d16 = st_d[pl.ds(j * 16, 16)]
        s16 = st_s[pl.ds(j * 16, 16)]
        m = (d16 >= lo) & (d16 < lo + WIN)
        mi = m.astype(jnp.int32)
        pos = cnt2 + plsc.cumsum(mi) - 1
        plsc.store_scatter(ls, [pos], s16, mask=m)
        plsc.store_scatter(ld, [pos], d16, mask=m)
        return cnt2 + jnp.sum(mi)

    cnt2 = lax.fori_loop(0, E2 // 16, wbody, jnp.int32(0))
    plsc.store_scatter(ls, [cnt2 + _iota16()], jnp.zeros((16,), jnp.int32))
    plsc.store_scatter(ld, [cnt2 + _iota16()], jnp.full((16,), lo, jnp.int32))

    def pbody(ci, carry):
        off = ci * 16
        s16 = ls[pl.ds(off, 16)]
        d16 = ld[pl.ds(off, 16)]
        sidx[...] = s16
        pltpu.sync_copy(hs2.at[sidx], h2buf)
        a_s = plsc.load_gather(a2_l, [s16 * 2])
        a_d = plsc.load_gather(a2_l, [d16 * 2 + 1])
        al = a_s + a_d
        al = jnp.where(al > 0, al, al * 0.2)
        eabuf[...] = jnp.exp(al)
        dloc = d16 - lo

        def ebody(e, carry2):
            dbase = jnp.sum(jnp.where(_iota16() == e, dloc, 0)) * ACC2W
            bc = _splat(eabuf, e)
            for g in range(C2 // 16):
                val = h2buf[e, pl.ds(g * 16, 16)] * bc
                plsc.addupdate(acc.at[pl.ds(dbase + g * 16, 16)], val)
            plsc.addupdate(
                acc.at[pl.ds(dbase + C2, 16)],
                jnp.where(_iota16() == 0, bc, jnp.zeros((16,), jnp.float32)))
            return carry2

        lax.fori_loop(0, jnp.minimum(cnt2 - off, 16), ebody, jnp.int32(0))
        return carry

    lax.fori_loop(0, (cnt2 + 15) // 16, pbody, jnp.int32(0))
    pltpu.sync_copy(acc, out_acc.at[pl.ds(w * WIN * ACC2W, WIN * ACC2W)])


def _scc(src2, dst2, a2f, hs2):
    mesh = plsc.VectorSubcoreMesh(core_axis_name="c", subcore_axis_name="s", num_cores=2, num_subcores=16)
    f = pl.kernel(
        _scc_body,
        out_type=jax.ShapeDtypeStruct((B * ACC2W,), jnp.float32),
        mesh=mesh,
        compiler_params=pltpu.CompilerParams(needs_layout_passes=False),
        scratch_types=[
            pltpu.VMEM((E2,), jnp.int32),
            pltpu.VMEM((E2,), jnp.int32),
            pltpu.VMEM((E2 + 16,), jnp.int32),
            pltpu.VMEM((E2 + 16,), jnp.int32),
            pltpu.VMEM((2 * B,), jnp.float32),
            pltpu.VMEM((16, C2), jnp.float32),
            pltpu.VMEM((16,), jnp.float32),
            pltpu.VMEM((16,), jnp.int32),
            pltpu.VMEM((16,), jnp.int32),
            pltpu.VMEM((WIN * ACC2W,), jnp.float32),
        ],
    )
    return f(src2, dst2, a2f, hs2)


def _tc3_body(raw_ref, b2_ref, wl_ref, bl_ref, wp_ref, bp_ref, o_ref):
    raw = raw_ref[...]
    den = raw[:, C2:C2 + 1]
    h2 = raw[:, :C2] / (den + 1e-16) + b2_ref[...]
    z = jnp.maximum(
        jnp.dot(h2, wl_ref[...], preferred_element_type=jnp.float32)
        + bl_ref[...], 0.0)
    o_ref[...] = (jnp.dot(z, wp_ref[...], preferred_element_type=jnp.float32)
                  + bp_ref[...])


def _tc3(out2raw, b2r, Wl, blr, Wp, bpr):
    return pl.pallas_call(
        _tc3_body,
        out_shape=jax.ShapeDtypeStruct((B, NUM_CLASSES), jnp.float32),
    )(out2raw, b2r, Wl, blr, Wp, bpr)


def kernel(x, edge_index1, edge_index2, W1, att_src1, att_dst1, b1, W2,
           att_src2, att_dst2, b2, Wl, bl, Wp, bp):
    xs = x[:N1]
    avs1 = att_src1.reshape(1, H1 * C1)
    avd1 = att_dst1.reshape(1, H1 * C1)
    hs, adst = _tc1(xs, W1, avs1, avd1)
    cs, cd, counts = _sca(edge_index1[0], edge_index1[1])
    out1raw = _scb(cs, cd, counts, adst, hs)
    hs2, a2I = _tc2(out1raw.reshape(B, ACC1W), b1.reshape(1, -1), W2,
                    att_src2.reshape(1, -1), att_dst2.reshape(1, -1))
    out2raw = _scc(edge_index2[0], edge_index2[1], a2I.reshape(-1), hs2)
    return _tc3(out2raw.reshape(B, ACC2W), b2.reshape(1, -1), Wl,
                bl.reshape(1, -1), Wp, bp.reshape(1, -1))


# speedup vs baseline: 50.3188x; 1.0359x over previous
import jax
import jax.numpy as jnp
from jax import lax
from jax.experimental import pallas as pl
from jax.experimental.pallas import tpu as pltpu
from jax.experimental.pallas import tpu_sc as plsc

N1 = 16384; B = 1024
E1 = 262144; E2 = 16384
H1 = 8; C1 = 256; C2 = 256; NUM_CLASSES = 1000

NW = 32
EPW1 = E1 // NW
EPW2 = E2 // NW
WIN = B // NW
CROW = 10240
CPAD = EPW1 + 16
ACC1W = 2064
HSW = 2176
ACC2W = 272
SENT = B


def _iota16():
    return lax.broadcasted_iota(jnp.int32, (16,), 0)


def _splat(buf, i):
    return plsc.load_gather(buf, [jnp.full((16,), i, jnp.int32)])


def _tc1_body(x_ref, w1_ref, avs_ref, avd_ref, hs_ref, adst_ref):
    i = pl.program_id(0)
    acc = jnp.dot(x_ref[...], w1_ref[...], preferred_element_type=jnp.float32)
    ts = acc * avs_ref[...]
    asrc = jnp.concatenate(
        [ts[:, h * C1:(h + 1) * C1].sum(axis=-1, keepdims=True)
         for h in range(H1)], axis=1)
    hs_ref[...] = jnp.concatenate(
        [acc, asrc,
         jnp.zeros((acc.shape[0], HSW - H1 * C1 - H1), jnp.float32)], axis=1)

    @pl.when(i == 0)
    def _():
        td = acc * avd_ref[...]
        adst_ref[...] = jnp.concatenate(
            [td[:B, h * C1:(h + 1) * C1].sum(axis=-1, keepdims=True)
             for h in range(H1)], axis=1)


def _tc1(xs, W1, avs1, avd1):
    bm = 1024
    return pl.pallas_call(
        _tc1_body,
        grid=(N1 // bm,),
        in_specs=[
            pl.BlockSpec((bm, 256), lambda i: (i, 0)),
            pl.BlockSpec((256, H1 * C1), lambda i: (0, 0)),
            pl.BlockSpec((1, H1 * C1), lambda i: (0, 0)),
            pl.BlockSpec((1, H1 * C1), lambda i: (0, 0)),
        ],
        out_specs=[
            pl.BlockSpec((bm, HSW), lambda i: (i, 0)),
            pl.BlockSpec((B, H1), lambda i: (0, 0)),
        ],
        out_shape=[
            jax.ShapeDtypeStruct((N1, HSW), jnp.float32),
            jax.ShapeDtypeStruct((B, H1), jnp.float32),
        ],
    )(xs, W1, avs1, avd1)


def _sca_body(srcE, dstE, cs, cd, counts, st_s, st_d, ls, ld, snt, cbuf):
    c = lax.axis_index("c")
    s = lax.axis_index("s")
    w = s * 2 + c
    pltpu.sync_copy(srcE.at[pl.ds(w * EPW1, EPW1)], st_s)
    pltpu.sync_copy(dstE.at[pl.ds(w * EPW1, EPW1)], st_d)
    for k in range(2048 // 16):
        snt[pl.ds(k * 16, 16)] = jnp.full((16,), SENT, jnp.int32)

    def fbody(j, carry):
        ls[pl.ds(j * 16, 16)] = jnp.zeros((16,), jnp.int32)
        ld[pl.ds(j * 16, 16)] = jnp.full((16,), SENT, jnp.int32)
        return carry

    lax.fori_loop(0, CPAD // 16, fbody, jnp.int32(0))

    def cbody(j, cnt):
        d16 = st_d[pl.ds(j * 16, 16)]
        s16 = st_s[pl.ds(j * 16, 16)]
        m = d16 < B
        mi = m.astype(jnp.int32)
        pos = cnt + plsc.cumsum(mi) - 1
        plsc.store_scatter(ls, [pos], s16, mask=m)
        plsc.store_scatter(ld, [pos], d16, mask=m)
        return cnt + jnp.sum(mi)

    cnt = lax.fori_loop(0, EPW1 // 16, cbody, jnp.int32(0))
    cntp = ((cnt + 15) // 16) * 16
    pltpu.sync_copy(snt, cs.at[pl.ds(w * CROW + 8192, 2048)])
    pltpu.sync_copy(snt, cd.at[pl.ds(w * CROW + 8192, 2048)])
    pltpu.sync_copy(ls, cs.at[pl.ds(w * CROW, CPAD)])
    pltpu.sync_copy(ld, cd.at[pl.ds(w * CROW, CPAD)])
    cbuf[...] = jnp.full((16,), cntp, jnp.int32)
    pltpu.sync_copy(cbuf, counts.at[pl.ds(w * 16, 16)])


def _sca(src1, dst1):
    mesh = plsc.VectorSubcoreMesh(core_axis_name="c", subcore_axis_name="s", num_cores=2, num_subcores=16)
    f = pl.kernel(
        _sca_body,
        out_type=[
            jax.ShapeDtypeStruct((NW * CROW,), jnp.int32),
            jax.ShapeDtypeStruct((NW * CROW,), jnp.int32),
            jax.ShapeDtypeStruct((NW * 16,), jnp.int32),
        ],
        mesh=mesh,
        compiler_params=pltpu.CompilerParams(needs_layout_passes=False),
        scratch_types=[
            pltpu.VMEM((EPW1,), jnp.int32),
            pltpu.VMEM((EPW1,), jnp.int32),
            pltpu.VMEM((CPAD,), jnp.int32),
            pltpu.VMEM((CPAD,), jnp.int32),
            pltpu.VMEM((2048,), jnp.int32),
            pltpu.VMEM((16,), jnp.int32),
        ],
    )
    return f(src1, dst1)


def _scb_body(cs, cd, counts, adst, hs, out_acc,
              st_s0, st_d0, st_s1, st_d1, ls, ld, cnts_l, adst_w, hsbuf,
              eabuf, sidx, acc, sem_s0, sem_d0, sem_s1, sem_d1):
    c = lax.axis_index("c")
    s = lax.axis_index("s")
    w = s * 2 + c
    lo = w * WIN

    def zbody(j, carry):
        for u in range(8):
            acc[pl.ds(j * 128 + u * 16, 16)] = jnp.zeros((16,), jnp.float32)
        return carry

    lax.fori_loop(0, (WIN * ACC1W) // 128, zbody, jnp.int32(0))
    pltpu.sync_copy(counts, cnts_l)
    pltpu.sync_copy(adst.at[pl.ds(lo, WIN)], adst_w)

    def process_chunk(s16, d16, nv):
        vmask = _iota16() < nv
        s16 = jnp.where(vmask, s16, 0)
        d16 = jnp.where(vmask, d16, lo)
        sidx[...] = s16
        pltpu.sync_copy(hs.at[sidx], hsbuf)
        dloc = d16 - lo
        for h in range(H1):
            a_s = plsc.load_gather(
                hsbuf, [_iota16(), jnp.full((16,), H1 * C1 + h, jnp.int32)])
            a_d = plsc.load_gather(
                adst_w, [dloc, jnp.full((16,), h, jnp.int32)])
            al = a_s + a_d
            al = jnp.where(al > 0, al, al * 0.2)
            eabuf[pl.ds(h * 16, 16)] = jnp.exp(al)

        def ebody(e, carry):
            dbase = jnp.sum(jnp.where(_iota16() == e, dloc, 0)) * ACC1W
            tail = jnp.zeros((16,), jnp.float32)
            for h in range(H1):
                bc = _splat(eabuf, h * 16 + e)
                tail = jnp.where(_iota16() == h, bc, tail)
                for g in range(C1 // 16):
                    o = h * C1 + g * 16
                    val = hsbuf[e, pl.ds(o, 16)] * bc
                    plsc.addupdate(acc.at[pl.ds(dbase + o, 16)], val)
            plsc.addupdate(acc.at[pl.ds(dbase + H1 * C1, 16)], tail)
            return carry

        lax.fori_loop(0, nv, ebody, jnp.int32(0))

    def scan_stage(st_s, st_d, nj):
        def wbody(j, cnt2):
            d16 = st_d[pl.ds(j * 16, 16)]
            s16 = st_s[pl.ds(j * 16, 16)]
            m = (d16 >= lo) & (d16 < lo + WIN)
            mi = m.astype(jnp.int32)
            pos = cnt2 + plsc.cumsum(mi) - 1
            plsc.store_scatter(ls, [pos], s16, mask=m)
            plsc.store_scatter(ld, [pos], d16, mask=m)
            return cnt2 + jnp.sum(mi)

        cnt2 = lax.fori_loop(0, nj, wbody, jnp.int32(0))
        plsc.store_scatter(ls, [cnt2 + _iota16()],
                           jnp.zeros((16,), jnp.int32))
        plsc.store_scatter(ld, [cnt2 + _iota16()],
                           jnp.full((16,), lo, jnp.int32))

        def pbody(ci, carry3):
            off = ci * 16
            process_chunk(ls[pl.ds(off, 16)], ld[pl.ds(off, 16)],
                          jnp.minimum(cnt2 - off, 16))
            return carry3

        lax.fori_loop(0, (cnt2 + 15) // 16, pbody, jnp.int32(0))

    def do_row(r, st_s, st_d):
        cr = jnp.max(cnts_l[pl.ds(r * 16, 16)])
        nst = (cr + 2047) // 2048
        scan_stage(st_s, st_d, (jnp.minimum(cr, 2048) + 15) // 16)

        def extra(t, carry2):
            pltpu.sync_copy(cs.at[pl.ds(r * CROW + (t + 1) * 2048, 2048)],
                            st_s)
            pltpu.sync_copy(cd.at[pl.ds(r * CROW + (t + 1) * 2048, 2048)],
                            st_d)
            scan_stage(st_s, st_d,
                       (jnp.minimum(cr - (t + 1) * 2048, 2048) + 15) // 16)
            return carry2

        lax.fori_loop(0, nst - 1, extra, jnp.int32(0))

    def start_row(r, st_s, st_d, sem_s, sem_d):
        rr = jnp.minimum(r, NW - 1)
        pltpu.async_copy(cs.at[pl.ds(rr * CROW, 2048)], st_s, sem_s)
        pltpu.async_copy(cd.at[pl.ds(rr * CROW, 2048)], st_d, sem_d)

    def wait_row(st_s, st_d, sem_s, sem_d):
        pltpu.make_async_copy(cs.at[pl.ds(0, 2048)], st_s, sem_s).wait()
        pltpu.make_async_copy(cd.at[pl.ds(0, 2048)], st_d, sem_d).wait()

    start_row(jnp.int32(0), st_s0, st_d0, sem_s0, sem_d0)

    def pair_loop(i, carry):
        r0 = i * 2
        wait_row(st_s0, st_d0, sem_s0, sem_d0)
        start_row(r0 + 1, st_s1, st_d1, sem_s1, sem_d1)
        do_row(r0, st_s0, st_d0)
        wait_row(st_s1, st_d1, sem_s1, sem_d1)
        start_row(r0 + 2, st_s0, st_d0, sem_s0, sem_d0)
        do_row(r0 + 1, st_s1, st_d1)
        return carry

    lax.fori_loop(0, NW // 2, pair_loop, jnp.int32(0))
    wait_row(st_s0, st_d0, sem_s0, sem_d0)
    pltpu.sync_copy(acc, out_acc.at[pl.ds(w * WIN * ACC1W, WIN * ACC1W)])


def _scb(cs, cd, counts, adstT, hsT):
    mesh = plsc.VectorSubcoreMesh(core_axis_name="c", subcore_axis_name="s", num_cores=2, num_subcores=16)
    f = pl.kernel(
        _scb_body,
        out_type=jax.ShapeDtypeStruct((B * ACC1W,), jnp.float32),
        mesh=mesh,
        compiler_params=pltpu.CompilerParams(needs_layout_passes=False),
        scratch_types=[
            pltpu.VMEM((2048,), jnp.int32),
            pltpu.VMEM((2048,), jnp.int32),
            pltpu.VMEM((2048,), jnp.int32),
            pltpu.VMEM((2048,), jnp.int32),
            pltpu.VMEM((2064 + 16,), jnp.int32),
            pltpu.VMEM((2064 + 16,), jnp.int32),
            pltpu.VMEM((NW * 16,), jnp.int32),
            pltpu.VMEM((WIN, H1), jnp.float32),
            pltpu.VMEM((16, HSW), jnp.float32),
            pltpu.VMEM((H1 * 16,), jnp.float32),
            pltpu.VMEM((16,), jnp.int32),
            pltpu.VMEM((WIN * ACC1W,), jnp.float32),
            pltpu.SemaphoreType.DMA,
            pltpu.SemaphoreType.DMA,
            pltpu.SemaphoreType.DMA,
            pltpu.SemaphoreType.DMA,
        ],
    )
    return f(cs, cd, counts, adstT, hsT)


def _tc2_body(raw_ref, b1_ref, w2_ref, avs2_ref, avd2_ref, hs2_ref, a2_ref):
    raw = raw_ref[...]
    parts = []
    for h in range(H1):
        num = raw[:, h * C1:(h + 1) * C1]
        den = raw[:, H1 * C1 + h:H1 * C1 + h + 1]
        parts.append(num / (den + 1e-16))
    out1 = jnp.concatenate(parts, axis=1)
    hmat = jnp.maximum(out1 + b1_ref[...], 0.0)
    hs2 = jnp.dot(hmat, w2_ref[...], preferred_element_type=jnp.float32)
    hs2_ref[...] = hs2
    t1 = (hs2 * avs2_ref[...]).sum(axis=-1, keepdims=True)
    t2 = (hs2 * avd2_ref[...]).sum(axis=-1, keepdims=True)
    a2_ref[...] = jnp.concatenate([t1, t2], axis=1)


def _tc2(out1raw, b1r, W2, avs2, avd2):
    return pl.pallas_call(
        _tc2_body,
        out_shape=[
            jax.ShapeDtypeStruct((B, C2), jnp.float32),
            jax.ShapeDtypeStruct((B, 2), jnp.float32),
        ],
    )(out1raw, b1r, W2, avs2, avd2)


def _scc_body(srcE, dstE, a2f, hs2, out_acc,
              st_s, st_d, ls, ld, a2_l, h2buf, eabuf, didx, sidx, acc):
    c = lax.axis_index("c")
    s = lax.axis_index("s")
    w = s * 2 + c
    lo = w * WIN

    def zbody(j, carry):
        acc[pl.ds(j * 16, 16)] = jnp.zeros((16,), jnp.float32)
        return carry

    lax.fori_loop(0, (WIN * ACC2W) // 16, zbody, jnp.int32(0))
    pltpu.sync_copy(srcE, st_s)
    pltpu.sync_copy(dstE, st_d)
    pltpu.sync_copy(a2f, a2_l)

    def wbody(j, cnt2):
        d16 = st_d[pl.ds(j * 16, 16)]
        s16 = st_s[pl.ds(j * 16, 16)]
        m = (d16 >= lo) & (d16 < lo + WIN)
        mi = m.astype(jnp.int32)
        pos = cnt2 + plsc.cumsum(mi) - 1
        plsc.store_scatter(ls, [pos], s16, mask=m)
        plsc.store_scatter(ld, [pos], d16, mask=m)
        return cnt2 + jnp.sum(mi)

    cnt2 = lax.fori_loop(0, E2 // 16, wbody, jnp.int32(0))
    plsc.store_scatter(ls, [cnt2 + _iota16()], jnp.zeros((16,), jnp.int32))
    plsc.store_scatter(ld, [cnt2 + _iota16()], jnp.full((16,), lo, jnp.int32))

    def pbody(ci, carry):
        off = ci * 16
        s16 = ls[pl.ds(off, 16)]
        d16 = ld[pl.ds(off, 16)]
        sidx[...] = s16
        pltpu.sync_copy(hs2.at[sidx], h2buf)
        a_s = plsc.load_gather(a2_l, [s16 * 2])
        a_d = plsc.load_gather(a2_l, [d16 * 2 + 1])
        al = a_s + a_d
        al = jnp.where(al > 0, al, al * 0.2)
        eabuf[...] = jnp.exp(al)
        dloc = d16 - lo

        def ebody(e, carry2):
            dbase = jnp.sum(jnp.where(_iota16() == e, dloc, 0)) * ACC2W
            bc = _splat(eabuf, e)
            for g in range(C2 // 16):
                val = h2buf[e, pl.ds(g * 16, 16)] * bc
                plsc.addupdate(acc.at[pl.ds(dbase + g * 16, 16)], val)
            plsc.addupdate(
                acc.at[pl.ds(dbase + C2, 16)],
                jnp.where(_iota16() == 0, bc, jnp.zeros((16,), jnp.float32)))
            return carry2

        lax.fori_loop(0, jnp.minimum(cnt2 - off, 16), ebody, jnp.int32(0))
        return carry

    lax.fori_loop(0, (cnt2 + 15) // 16, pbody, jnp.int32(0))
    pltpu.sync_copy(acc, out_acc.at[pl.ds(w * WIN * ACC2W, WIN * ACC2W)])


def _scc(src2, dst2, a2f, hs2):
    mesh = plsc.VectorSubcoreMesh(core_axis_name="c", subcore_axis_name="s", num_cores=2, num_subcores=16)
    f = pl.kernel(
        _scc_body,
        out_type=jax.ShapeDtypeStruct((B * ACC2W,), jnp.float32),
        mesh=mesh,
        compiler_params=pltpu.CompilerParams(needs_layout_passes=False),
        scratch_types=[
            pltpu.VMEM((E2,), jnp.int32),
            pltpu.VMEM((E2,), jnp.int32),
            pltpu.VMEM((E2 + 16,), jnp.int32),
            pltpu.VMEM((E2 + 16,), jnp.int32),
            pltpu.VMEM((2 * B,), jnp.float32),
            pltpu.VMEM((16, C2), jnp.float32),
            pltpu.VMEM((16,), jnp.float32),
            pltpu.VMEM((16,), jnp.int32),
            pltpu.VMEM((16,), jnp.int32),
            pltpu.VMEM((WIN * ACC2W,), jnp.float32),
        ],
    )
    return f(src2, dst2, a2f, hs2)


def _tc3_body(raw_ref, b2_ref, wl_ref, bl_ref, wp_ref, bp_ref, o_ref):
    raw = raw_ref[...]
    den = raw[:, C2:C2 + 1]
    h2 = raw[:, :C2] / (den + 1e-16) + b2_ref[...]
    z = jnp.maximum(
        jnp.dot(h2, wl_ref[...], preferred_element_type=jnp.float32)
        + bl_ref[...], 0.0)
    o_ref[...] = (jnp.dot(z, wp_ref[...], preferred_element_type=jnp.float32)
                  + bp_ref[...])


def _tc3(out2raw, b2r, Wl, blr, Wp, bpr):
    return pl.pallas_call(
        _tc3_body,
        out_shape=jax.ShapeDtypeStruct((B, NUM_CLASSES), jnp.float32),
    )(out2raw, b2r, Wl, blr, Wp, bpr)


def kernel(x, edge_index1, edge_index2, W1, att_src1, att_dst1, b1, W2,
           att_src2, att_dst2, b2, Wl, bl, Wp, bp):
    xs = x[:N1]
    avs1 = att_src1.reshape(1, H1 * C1)
    avd1 = att_dst1.reshape(1, H1 * C1)
    hs, adst = _tc1(xs, W1, avs1, avd1)
    cs, cd, counts = _sca(edge_index1[0], edge_index1[1])
    out1raw = _scb(cs, cd, counts, adst, hs)
    hs2, a2I = _tc2(out1raw.reshape(B, ACC1W), b1.reshape(1, -1), W2,
                    att_src2.reshape(1, -1), att_dst2.reshape(1, -1))
    out2raw = _scc(edge_index2[0], edge_index2[1], a2I.reshape(-1), hs2)
    return _tc3(out2raw.reshape(B, ACC2W), b2.reshape(1, -1), Wl,
                bl.reshape(1, -1), Wp, bp.reshape(1, -1))
